# baseline (speedup 1.0000x reference)
import jax
import jax.numpy as jnp
from jax.experimental import pallas as pl
from jax.experimental.pallas import tpu as pltpu
from jax.experimental.pallas import tpu_sc as plsc

_EPS = 1e-5
_GW = 128
_IPR = 32


def _sc_gather(tables_flat, flat_idx):
    n = flat_idx.shape[1]
    d = tables_flat.shape[1]
    mesh = plsc.VectorSubcoreMesh(core_axis_name="core", subcore_axis_name="subcore")

    @pl.kernel(
        out_type=jax.ShapeDtypeStruct((n, d), tables_flat.dtype),
        mesh=mesh,
        compiler_params=pltpu.CompilerParams(use_tc_tiling_on_sc=False),
    )
    def gather_kernel(tab_hbm, idx_hbm, out_hbm):
        def body(i_vmem, o_vmem):
            pltpu.sync_copy(tab_hbm.at[i_vmem.at[0]], o_vmem)

        pltpu.emit_pipeline(
            body,
            grid=(n // _GW,),
            in_specs=[pl.BlockSpec((1, _GW), index_map=lambda i: (0, i))],
            out_specs=[pl.BlockSpec((_GW, d), index_map=lambda i: (i, 0))],
            core_axis_name=("core", "subcore"),
            dimension_semantics=(pltpu.PARALLEL,),
        )(idx_hbm, out_hbm)

    return gather_kernel(tables_flat, flat_idx)


_BLK = 2048


_BLK1 = 2048


def _layer1(xn, emb_rows, w1n, w1e, b1):
    b, h = xn.shape[0], w1n.shape[1]

    def body(xn_ref, e_ref, wn_ref, we_ref, b_ref, z_ref, s_ref, q_ref):
        z = jnp.dot(xn_ref[...], wn_ref[...], preferred_element_type=jnp.float32)
        z = z + jnp.dot(e_ref[...], we_ref[...], preferred_element_type=jnp.float32)
        z = z + b_ref[...]
        z_ref[...] = z

        @pl.when(pl.program_id(0) == 0)
        def _():
            s_ref[...] = jnp.zeros_like(s_ref)
            q_ref[...] = jnp.zeros_like(q_ref)

        s_ref[...] += jnp.sum(z, axis=0)
        q_ref[...] += jnp.sum(z * z, axis=0)

    return pl.pallas_call(
        body,
        grid=(b // _BLK1,),
        in_specs=[
            pl.BlockSpec((_BLK1, xn.shape[1]), lambda i: (i, 0)),
            pl.BlockSpec((_BLK1, emb_rows.shape[1]), lambda i: (i, 0)),
            pl.BlockSpec(w1n.shape, lambda i: (0, 0)),
            pl.BlockSpec(w1e.shape, lambda i: (0, 0)),
            pl.BlockSpec(b1.shape, lambda i: (0,)),
        ],
        out_specs=[
            pl.BlockSpec((_BLK1, h), lambda i: (i, 0)),
            pl.BlockSpec((h,), lambda i: (0,)),
            pl.BlockSpec((h,), lambda i: (0,)),
        ],
        out_shape=[
            jax.ShapeDtypeStruct((b, h), jnp.float32),
            jax.ShapeDtypeStruct((h,), jnp.float32),
            jax.ShapeDtypeStruct((h,), jnp.float32),
        ],
        compiler_params=pltpu.CompilerParams(
            dimension_semantics=("arbitrary",)
        ),
    )(xn, emb_rows, w1n, w1e, b1)


def _bn_relu_matmul(z, s, q, g, beta, w, bias):
    b, h_out = z.shape[0], w.shape[1]

    def body(z_ref, s_ref, q_ref, g_ref, be_ref, w_ref, b_ref, z2_ref, s2_ref, q2_ref):
        mu = s_ref[...] * (1.0 / b)
        var = q_ref[...] * (1.0 / b) - mu * mu
        a = g_ref[...] * jax.lax.rsqrt(var + _EPS)
        c = be_ref[...] - a * mu
        h = jnp.maximum(z_ref[...] * a + c, 0.0)
        z2 = jnp.dot(h, w_ref[...], preferred_element_type=jnp.float32) + b_ref[...]
        z2_ref[...] = z2

        @pl.when(pl.program_id(0) == 0)
        def _():
            s2_ref[...] = jnp.zeros_like(s2_ref)
            q2_ref[...] = jnp.zeros_like(q2_ref)

        s2_ref[...] += jnp.sum(z2, axis=0)
        q2_ref[...] += jnp.sum(z2 * z2, axis=0)

    return pl.pallas_call(
        body,
        grid=(b // _BLK,),
        in_specs=[
            pl.BlockSpec((_BLK, z.shape[1]), lambda i: (i, 0)),
            pl.BlockSpec(s.shape, lambda i: (0,)),
            pl.BlockSpec(q.shape, lambda i: (0,)),
            pl.BlockSpec(g.shape, lambda i: (0,)),
            pl.BlockSpec(beta.shape, lambda i: (0,)),
            pl.BlockSpec(w.shape, lambda i: (0, 0)),
            pl.BlockSpec(bias.shape, lambda i: (0,)),
        ],
        out_specs=[
            pl.BlockSpec((_BLK, h_out), lambda i: (i, 0)),
            pl.BlockSpec((h_out,), lambda i: (0,)),
            pl.BlockSpec((h_out,), lambda i: (0,)),
        ],
        out_shape=[
            jax.ShapeDtypeStruct((b, h_out), jnp.float32),
            jax.ShapeDtypeStruct((h_out,), jnp.float32),
            jax.ShapeDtypeStruct((h_out,), jnp.float32),
        ],
        compiler_params=pltpu.CompilerParams(
            dimension_semantics=("arbitrary",)
        ),
    )(z, s, q, g, beta, w, bias)


def _final(z, s, q, g, beta, w4, b4):
    b = z.shape[0]

    def body(z_ref, s_ref, q_ref, g_ref, be_ref, w_ref, b_ref, o_ref):
        mu = s_ref[...] * (1.0 / b)
        var = q_ref[...] * (1.0 / b) - mu * mu
        a = g_ref[...] * jax.lax.rsqrt(var + _EPS)
        c = be_ref[...] - a * mu
        h = jnp.maximum(z_ref[...] * a + c, 0.0)
        logit = jnp.dot(h, w_ref[...], preferred_element_type=jnp.float32) + b_ref[...]
        o_ref[...] = jax.nn.sigmoid(logit)

    return pl.pallas_call(
        body,
        grid=(b // _BLK,),
        in_specs=[
            pl.BlockSpec((_BLK, z.shape[1]), lambda i: (i, 0)),
            pl.BlockSpec(s.shape, lambda i: (0,)),
            pl.BlockSpec(q.shape, lambda i: (0,)),
            pl.BlockSpec(g.shape, lambda i: (0,)),
            pl.BlockSpec(beta.shape, lambda i: (0,)),
            pl.BlockSpec(w4.shape, lambda i: (0, 0)),
            pl.BlockSpec(b4.shape, lambda i: (0,)),
        ],
        out_specs=pl.BlockSpec((_BLK, 1), lambda i: (i, 0)),
        out_shape=jax.ShapeDtypeStruct((b, 1), jnp.float32),
        compiler_params=pltpu.CompilerParams(
            dimension_semantics=("arbitrary",)
        ),
    )(z, s, q, g, beta, w4, b4)


def kernel(X_numerical, X_categorical, tables, W1, b1, g1, beta1, W2, b2, g2, beta2, W3, b3, g3, beta3, W4, b4):
    f, v, d = tables.shape
    b = X_numerical.shape[0]
    nnum = X_numerical.shape[1]

    tables_flat = tables.reshape(f * v, d)
    flat2 = (X_categorical.astype(jnp.int32)
             + (jnp.arange(f, dtype=jnp.int32) * v)[None, :])
    idx_pad = jnp.zeros((b, _IPR), jnp.int32).at[:, :f].set(flat2)
    flat_idx = idx_pad.reshape(1, b * _IPR)

    emb_rows = _sc_gather(tables_flat, flat_idx)
    emb = emb_rows.reshape(b, _IPR * d)

    w1n = W1[:nnum]
    w1e = jnp.zeros((_IPR * d, W1.shape[1]), jnp.float32).at[: f * d].set(W1[nnum:])
    z1, s1, q1 = _layer1(X_numerical, emb, w1n, w1e, b1)
    z2, s2, q2 = _bn_relu_matmul(z1, s1, q1, g1, beta1, W2, b2)
    z3, s3, q3 = _bn_relu_matmul(z2, s2, q2, g2, beta2, W3, b3)
    out = _final(z3, s3, q3, g3, beta3, W4, b4)
    return out.reshape(b)


# speedup vs baseline: 1.3425x; 1.3425x over previous
import jax
import jax.numpy as jnp
from jax.experimental import pallas as pl
from jax.experimental.pallas import tpu as pltpu
from jax.experimental.pallas import tpu_sc as plsc

_EPS = 1e-5
_GW = 128
_IPR = 32
_NPL = 4


def _sc_gather(tables_flat, flat_idx):
    n = flat_idx.shape[1]
    d = tables_flat.shape[1]
    mesh = plsc.VectorSubcoreMesh(core_axis_name="core", subcore_axis_name="subcore")

    @pl.kernel(
        out_type=jax.ShapeDtypeStruct((n, d), tables_flat.dtype),
        mesh=mesh,
        compiler_params=pltpu.CompilerParams(use_tc_tiling_on_sc=False),
    )
    def gather_kernel(tab_hbm, idx_hbm, out_hbm):
        def body(i_vmem, o_vmem):
            pltpu.sync_copy(tab_hbm.at[i_vmem.at[0]], o_vmem)

        pltpu.emit_pipeline(
            body,
            grid=(n // _GW,),
            in_specs=[pl.BlockSpec((1, _GW), index_map=lambda i: (0, i))],
            out_specs=[pl.BlockSpec((_GW, d), index_map=lambda i: (i, 0))],
            core_axis_name=("core", "subcore"),
            dimension_semantics=(pltpu.PARALLEL,),
        )(idx_hbm, out_hbm)

    return gather_kernel(tables_flat, flat_idx)


_BLK = 2048


_BLK1 = 2048


def _layer1(xn, emb_rows, w1n, w1e, b1):
    b, h = xn.shape[0], w1n.shape[1]

    def body(xn_ref, e_ref, wn_ref, we_ref, b_ref, z_ref, s_ref, q_ref):
        z = jnp.dot(xn_ref[...], wn_ref[...], preferred_element_type=jnp.float32)
        for j in range(_NPL):
            z = z + jnp.dot(e_ref[j], we_ref[j],
                            preferred_element_type=jnp.float32)
        z = z + b_ref[...]
        z_ref[...] = z

        @pl.when(pl.program_id(0) == 0)
        def _():
            s_ref[...] = jnp.zeros_like(s_ref)
            q_ref[...] = jnp.zeros_like(q_ref)

        s_ref[...] += jnp.sum(z, axis=0)
        q_ref[...] += jnp.sum(z * z, axis=0)

    return pl.pallas_call(
        body,
        grid=(b // _BLK1,),
        in_specs=[
            pl.BlockSpec((_BLK1, xn.shape[1]), lambda i: (i, 0)),
            pl.BlockSpec((_NPL, _BLK1, 128), lambda i: (0, i, 0)),
            pl.BlockSpec(w1n.shape, lambda i: (0, 0)),
            pl.BlockSpec(w1e.shape, lambda i: (0, 0, 0)),
            pl.BlockSpec(b1.shape, lambda i: (0,)),
        ],
        out_specs=[
            pl.BlockSpec((_BLK1, h), lambda i: (i, 0)),
            pl.BlockSpec((h,), lambda i: (0,)),
            pl.BlockSpec((h,), lambda i: (0,)),
        ],
        out_shape=[
            jax.ShapeDtypeStruct((b, h), jnp.float32),
            jax.ShapeDtypeStruct((h,), jnp.float32),
            jax.ShapeDtypeStruct((h,), jnp.float32),
        ],
        compiler_params=pltpu.CompilerParams(
            dimension_semantics=("arbitrary",)
        ),
    )(xn, emb_rows, w1n, w1e, b1)


def _bn_relu_matmul(z, s, q, g, beta, w, bias):
    b, h_out = z.shape[0], w.shape[1]

    def body(z_ref, s_ref, q_ref, g_ref, be_ref, w_ref, b_ref, z2_ref, s2_ref, q2_ref):
        mu = s_ref[...] * (1.0 / b)
        var = q_ref[...] * (1.0 / b) - mu * mu
        a = g_ref[...] * jax.lax.rsqrt(var + _EPS)
        c = be_ref[...] - a * mu
        h = jnp.maximum(z_ref[...] * a + c, 0.0)
        z2 = jnp.dot(h, w_ref[...], preferred_element_type=jnp.float32) + b_ref[...]
        z2_ref[...] = z2

        @pl.when(pl.program_id(0) == 0)
        def _():
            s2_ref[...] = jnp.zeros_like(s2_ref)
            q2_ref[...] = jnp.zeros_like(q2_ref)

        s2_ref[...] += jnp.sum(z2, axis=0)
        q2_ref[...] += jnp.sum(z2 * z2, axis=0)

    return pl.pallas_call(
        body,
        grid=(b // _BLK,),
        in_specs=[
            pl.BlockSpec((_BLK, z.shape[1]), lambda i: (i, 0)),
            pl.BlockSpec(s.shape, lambda i: (0,)),
            pl.BlockSpec(q.shape, lambda i: (0,)),
            pl.BlockSpec(g.shape, lambda i: (0,)),
            pl.BlockSpec(beta.shape, lambda i: (0,)),
            pl.BlockSpec(w.shape, lambda i: (0, 0)),
            pl.BlockSpec(bias.shape, lambda i: (0,)),
        ],
        out_specs=[
            pl.BlockSpec((_BLK, h_out), lambda i: (i, 0)),
            pl.BlockSpec((h_out,), lambda i: (0,)),
            pl.BlockSpec((h_out,), lambda i: (0,)),
        ],
        out_shape=[
            jax.ShapeDtypeStruct((b, h_out), jnp.float32),
            jax.ShapeDtypeStruct((h_out,), jnp.float32),
            jax.ShapeDtypeStruct((h_out,), jnp.float32),
        ],
        compiler_params=pltpu.CompilerParams(
            dimension_semantics=("arbitrary",)
        ),
    )(z, s, q, g, beta, w, bias)


def _final(z, s, q, g, beta, w4, b4):
    b = z.shape[0]

    def body(z_ref, s_ref, q_ref, g_ref, be_ref, w_ref, b_ref, o_ref):
        mu = s_ref[...] * (1.0 / b)
        var = q_ref[...] * (1.0 / b) - mu * mu
        a = g_ref[...] * jax.lax.rsqrt(var + _EPS)
        c = be_ref[...] - a * mu
        h = jnp.maximum(z_ref[...] * a + c, 0.0)
        logit = jnp.dot(h, w_ref[...], preferred_element_type=jnp.float32) + b_ref[...]
        o_ref[...] = jax.nn.sigmoid(logit)

    return pl.pallas_call(
        body,
        grid=(b // _BLK,),
        in_specs=[
            pl.BlockSpec((_BLK, z.shape[1]), lambda i: (i, 0)),
            pl.BlockSpec(s.shape, lambda i: (0,)),
            pl.BlockSpec(q.shape, lambda i: (0,)),
            pl.BlockSpec(g.shape, lambda i: (0,)),
            pl.BlockSpec(beta.shape, lambda i: (0,)),
            pl.BlockSpec(w4.shape, lambda i: (0, 0)),
            pl.BlockSpec(b4.shape, lambda i: (0,)),
        ],
        out_specs=pl.BlockSpec((_BLK, 1), lambda i: (i, 0)),
        out_shape=jax.ShapeDtypeStruct((b, 1), jnp.float32),
        compiler_params=pltpu.CompilerParams(
            dimension_semantics=("arbitrary",)
        ),
    )(z, s, q, g, beta, w4, b4)


def kernel(X_numerical, X_categorical, tables, W1, b1, g1, beta1, W2, b2, g2, beta2, W3, b3, g3, beta3, W4, b4):
    f, v, d = tables.shape
    b = X_numerical.shape[0]
    nnum = X_numerical.shape[1]

    tables_flat = tables.reshape(f * v, d)
    flat2 = (X_categorical.astype(jnp.int32)
             + (jnp.arange(f, dtype=jnp.int32) * v)[None, :])
    idx32 = jnp.concatenate([flat2, flat2[:, f - 6:]], axis=1)
    flat_idx = idx32.reshape(b, _NPL, 8).transpose(1, 0, 2).reshape(1, b * _IPR)

    emb_rows = _sc_gather(tables_flat, flat_idx)
    emb4 = emb_rows.reshape(_NPL, b, 8 * d)

    w1n = W1[:nnum]
    w1e_pad = jnp.zeros((_IPR * d, W1.shape[1]), jnp.float32).at[: f * d].set(W1[nnum:])
    w1e4 = w1e_pad.reshape(_NPL, 8 * d, W1.shape[1])
    z1, s1, q1 = _layer1(X_numerical, emb4, w1n, w1e4, b1)
    z2, s2, q2 = _bn_relu_matmul(z1, s1, q1, g1, beta1, W2, b2)
    z3, s3, q3 = _bn_relu_matmul(z2, s2, q2, g2, beta2, W3, b3)
    out = _final(z3, s3, q3, g3, beta3, W4, b4)
    return out.reshape(b)


# speedup vs baseline: 3.8928x; 2.8997x over previous
import jax
import jax.numpy as jnp
from jax.experimental import pallas as pl
from jax.experimental.pallas import tpu as pltpu
from jax.experimental.pallas import tpu_sc as plsc

_EPS = 1e-5
_GW = 128
_IPR = 32
_NPL = 4


_VB = 4096


def _table_planes(t416, v):
    def body(a_ref, b_ref, o_ref):
        j = pl.program_id(0)

        @pl.when(j < 3)
        def _():
            o_ref[0] = a_ref[...].T

        @pl.when(j == 3)
        def _():
            t = b_ref[...].T
            o_ref[0] = jnp.concatenate(
                [t, jnp.zeros((_VB, 96), jnp.float32)], axis=1)

    return pl.pallas_call(
        body,
        grid=(4, (v + _VB - 1) // _VB),
        in_specs=[
            pl.BlockSpec((128, _VB), lambda j, i: (jnp.minimum(j, 2), i)),
            pl.BlockSpec((32, _VB), lambda j, i: (12, i)),
        ],
        out_specs=pl.BlockSpec((1, _VB, 128), lambda j, i: (j, i, 0)),
        out_shape=jax.ShapeDtypeStruct((4, v, 128), jnp.float32),
        compiler_params=pltpu.CompilerParams(
            dimension_semantics=("arbitrary", "arbitrary")
        ),
    )(t416, t416)


def _sc_gather(tables_flat, flat_idx):
    n = flat_idx.shape[1]
    d = tables_flat.shape[1]
    mesh = plsc.VectorSubcoreMesh(core_axis_name="core", subcore_axis_name="subcore")

    @pl.kernel(
        out_type=jax.ShapeDtypeStruct((n, d), tables_flat.dtype),
        mesh=mesh,
        compiler_params=pltpu.CompilerParams(use_tc_tiling_on_sc=False),
    )
    def gather_kernel(tab_hbm, idx_hbm, out_hbm):
        def body(i_vmem, o_vmem):
            pltpu.sync_copy(tab_hbm.at[i_vmem.at[0]], o_vmem)

        pltpu.emit_pipeline(
            body,
            grid=(n // _GW,),
            in_specs=[pl.BlockSpec((1, _GW), index_map=lambda i: (0, i))],
            out_specs=[pl.BlockSpec((_GW, d), index_map=lambda i: (i, 0))],
            core_axis_name=("core", "subcore"),
            dimension_semantics=(pltpu.PARALLEL,),
        )(idx_hbm, out_hbm)

    return gather_kernel(tables_flat, flat_idx)


_BLK = 2048


_BLK1 = 2048


def _layer1(xn, emb_rows, w1n, w1e, b1):
    b, h = xn.shape[0], w1n.shape[1]

    def body(xn_ref, e_ref, wn_ref, we_ref, b_ref, z_ref, s_ref, q_ref):
        z = jnp.dot(xn_ref[...], wn_ref[...], preferred_element_type=jnp.float32)
        for j in range(_NPL):
            z = z + jnp.dot(e_ref[j], we_ref[j],
                            preferred_element_type=jnp.float32)
        z = z + b_ref[...]
        z_ref[...] = z

        @pl.when(pl.program_id(0) == 0)
        def _():
            s_ref[...] = jnp.zeros_like(s_ref)
            q_ref[...] = jnp.zeros_like(q_ref)

        s_ref[...] += jnp.sum(z, axis=0)
        q_ref[...] += jnp.sum(z * z, axis=0)

    return pl.pallas_call(
        body,
        grid=(b // _BLK1,),
        in_specs=[
            pl.BlockSpec((_BLK1, xn.shape[1]), lambda i: (i, 0)),
            pl.BlockSpec((_NPL, _BLK1, 128), lambda i: (0, i, 0)),
            pl.BlockSpec(w1n.shape, lambda i: (0, 0)),
            pl.BlockSpec(w1e.shape, lambda i: (0, 0, 0)),
            pl.BlockSpec(b1.shape, lambda i: (0,)),
        ],
        out_specs=[
            pl.BlockSpec((_BLK1, h), lambda i: (i, 0)),
            pl.BlockSpec((h,), lambda i: (0,)),
            pl.BlockSpec((h,), lambda i: (0,)),
        ],
        out_shape=[
            jax.ShapeDtypeStruct((b, h), jnp.float32),
            jax.ShapeDtypeStruct((h,), jnp.float32),
            jax.ShapeDtypeStruct((h,), jnp.float32),
        ],
        compiler_params=pltpu.CompilerParams(
            dimension_semantics=("arbitrary",)
        ),
    )(xn, emb_rows, w1n, w1e, b1)


def _bn_relu_matmul(z, s, q, g, beta, w, bias):
    b, h_out = z.shape[0], w.shape[1]

    def body(z_ref, s_ref, q_ref, g_ref, be_ref, w_ref, b_ref, z2_ref, s2_ref, q2_ref):
        mu = s_ref[...] * (1.0 / b)
        var = q_ref[...] * (1.0 / b) - mu * mu
        a = g_ref[...] * jax.lax.rsqrt(var + _EPS)
        c = be_ref[...] - a * mu
        h = jnp.maximum(z_ref[...] * a + c, 0.0)
        z2 = jnp.dot(h, w_ref[...], preferred_element_type=jnp.float32) + b_ref[...]
        z2_ref[...] = z2

        @pl.when(pl.program_id(0) == 0)
        def _():
            s2_ref[...] = jnp.zeros_like(s2_ref)
            q2_ref[...] = jnp.zeros_like(q2_ref)

        s2_ref[...] += jnp.sum(z2, axis=0)
        q2_ref[...] += jnp.sum(z2 * z2, axis=0)

    return pl.pallas_call(
        body,
        grid=(b // _BLK,),
        in_specs=[
            pl.BlockSpec((_BLK, z.shape[1]), lambda i: (i, 0)),
            pl.BlockSpec(s.shape, lambda i: (0,)),
            pl.BlockSpec(q.shape, lambda i: (0,)),
            pl.BlockSpec(g.shape, lambda i: (0,)),
            pl.BlockSpec(beta.shape, lambda i: (0,)),
            pl.BlockSpec(w.shape, lambda i: (0, 0)),
            pl.BlockSpec(bias.shape, lambda i: (0,)),
        ],
        out_specs=[
            pl.BlockSpec((_BLK, h_out), lambda i: (i, 0)),
            pl.BlockSpec((h_out,), lambda i: (0,)),
            pl.BlockSpec((h_out,), lambda i: (0,)),
        ],
        out_shape=[
            jax.ShapeDtypeStruct((b, h_out), jnp.float32),
            jax.ShapeDtypeStruct((h_out,), jnp.float32),
            jax.ShapeDtypeStruct((h_out,), jnp.float32),
        ],
        compiler_params=pltpu.CompilerParams(
            dimension_semantics=("arbitrary",)
        ),
    )(z, s, q, g, beta, w, bias)


def _final(z, s, q, g, beta, w4, b4):
    b = z.shape[0]

    def body(z_ref, s_ref, q_ref, g_ref, be_ref, w_ref, b_ref, o_ref):
        mu = s_ref[...] * (1.0 / b)
        var = q_ref[...] * (1.0 / b) - mu * mu
        a = g_ref[...] * jax.lax.rsqrt(var + _EPS)
        c = be_ref[...] - a * mu
        h = jnp.maximum(z_ref[...] * a + c, 0.0)
        logit = jnp.dot(h, w_ref[...], preferred_element_type=jnp.float32) + b_ref[...]
        o_ref[...] = jax.nn.sigmoid(logit)

    return pl.pallas_call(
        body,
        grid=(b // _BLK,),
        in_specs=[
            pl.BlockSpec((_BLK, z.shape[1]), lambda i: (i, 0)),
            pl.BlockSpec(s.shape, lambda i: (0,)),
            pl.BlockSpec(q.shape, lambda i: (0,)),
            pl.BlockSpec(g.shape, lambda i: (0,)),
            pl.BlockSpec(beta.shape, lambda i: (0,)),
            pl.BlockSpec(w4.shape, lambda i: (0, 0)),
            pl.BlockSpec(b4.shape, lambda i: (0,)),
        ],
        out_specs=pl.BlockSpec((_BLK, 1), lambda i: (i, 0)),
        out_shape=jax.ShapeDtypeStruct((b, 1), jnp.float32),
        compiler_params=pltpu.CompilerParams(
            dimension_semantics=("arbitrary",)
        ),
    )(z, s, q, g, beta, w4, b4)


def kernel(X_numerical, X_categorical, tables, W1, b1, g1, beta1, W2, b2, g2, beta2, W3, b3, g3, beta3, W4, b4):
    f, v, d = tables.shape
    b = X_numerical.shape[0]
    nnum = X_numerical.shape[1]

    t416 = tables.transpose(0, 2, 1).reshape(f * d, v)
    planes = _table_planes(t416, v)
    tables_flat = planes.reshape(_NPL * v * 8, d)

    s_arr = jnp.arange(_IPR, dtype=jnp.int32)
    f_src = jnp.where(s_arr < f, s_arr, s_arr - 6)
    vcat = X_categorical.astype(jnp.int32)[:, f_src]
    idx32 = (s_arr // 8) * (8 * v) + vcat * 8 + (s_arr % 8)
    flat_idx = idx32.reshape(b, _NPL, 8).transpose(1, 0, 2).reshape(1, b * _IPR)

    emb_rows = _sc_gather(tables_flat, flat_idx)
    emb4 = emb_rows.reshape(_NPL, b, 8 * d)

    w1n = W1[:nnum]
    w1e_pad = jnp.zeros((_IPR * d, W1.shape[1]), jnp.float32).at[: f * d].set(W1[nnum:])
    w1e4 = w1e_pad.reshape(_NPL, 8 * d, W1.shape[1])
    z1, s1, q1 = _layer1(X_numerical, emb4, w1n, w1e4, b1)
    z2, s2, q2 = _bn_relu_matmul(z1, s1, q1, g1, beta1, W2, b2)
    z3, s3, q3 = _bn_relu_matmul(z2, s2, q2, g2, beta2, W3, b3)
    out = _final(z3, s3, q3, g3, beta3, W4, b4)
    return out.reshape(b)


# speedup vs baseline: 4.0811x; 1.0484x over previous
import jax
import jax.numpy as jnp
from jax.experimental import pallas as pl
from jax.experimental.pallas import tpu as pltpu
from jax.experimental.pallas import tpu_sc as plsc

_EPS = 1e-5
_GW = 128
_IPR = 32
_NPL = 4


_VB = 4096


def _table_planes(t416, v):
    def body(a_ref, b_ref, o_ref):
        j = pl.program_id(0)

        @pl.when(j < 3)
        def _():
            o_ref[0] = a_ref[...].T

        @pl.when(j == 3)
        def _():
            t = b_ref[...].T
            o_ref[0] = jnp.concatenate(
                [t, jnp.zeros((_VB, 96), jnp.float32)], axis=1)

    return pl.pallas_call(
        body,
        grid=(4, (v + _VB - 1) // _VB),
        in_specs=[
            pl.BlockSpec((128, _VB), lambda j, i: (jnp.minimum(j, 2), i)),
            pl.BlockSpec((32, _VB), lambda j, i: (12, i)),
        ],
        out_specs=pl.BlockSpec((1, _VB, 128), lambda j, i: (j, i, 0)),
        out_shape=jax.ShapeDtypeStruct((4, v, 128), jnp.float32),
        compiler_params=pltpu.CompilerParams(
            dimension_semantics=("arbitrary", "arbitrary")
        ),
    )(t416, t416)


def _sc_gather(tables_flat, flat_idx):
    n = flat_idx.shape[1]
    d = tables_flat.shape[1]
    mesh = plsc.VectorSubcoreMesh(core_axis_name="core", subcore_axis_name="subcore")

    @pl.kernel(
        out_type=jax.ShapeDtypeStruct((n, d), tables_flat.dtype),
        mesh=mesh,
        compiler_params=pltpu.CompilerParams(use_tc_tiling_on_sc=False),
    )
    def gather_kernel(tab_hbm, idx_hbm, out_hbm):
        def body(i_vmem, o_vmem):
            pltpu.sync_copy(tab_hbm.at[i_vmem.at[0]], o_vmem)

        pltpu.emit_pipeline(
            body,
            grid=(n // _GW,),
            in_specs=[pl.BlockSpec((1, _GW), index_map=lambda i: (0, i))],
            out_specs=[pl.BlockSpec((_GW, d), index_map=lambda i: (i, 0))],
            core_axis_name=("core", "subcore"),
            dimension_semantics=(pltpu.PARALLEL,),
        )(idx_hbm, out_hbm)

    return gather_kernel(tables_flat, flat_idx)


_BLK = 2048


_BLK1 = 2048


def _layer1(xn, emb_rows, w1n, w1e, b1):
    b, h = xn.shape[0], w1n.shape[1]

    def body(xn_ref, e_ref, wn_ref, we_ref, b_ref, z_ref, s_ref, q_ref):
        z = jnp.dot(xn_ref[...], wn_ref[...], preferred_element_type=jnp.float32)
        for j in range(_NPL):
            z = z + jnp.dot(e_ref[j], we_ref[j],
                            preferred_element_type=jnp.float32)
        z = z + b_ref[...]
        z_ref[...] = z

        @pl.when(pl.program_id(0) == 0)
        def _():
            s_ref[...] = jnp.zeros_like(s_ref)
            q_ref[...] = jnp.zeros_like(q_ref)

        s_ref[...] += jnp.sum(z, axis=0)
        q_ref[...] += jnp.sum(z * z, axis=0)

    return pl.pallas_call(
        body,
        grid=(b // _BLK1,),
        in_specs=[
            pl.BlockSpec((_BLK1, xn.shape[1]), lambda i: (i, 0)),
            pl.BlockSpec((_NPL, _BLK1, 128), lambda i: (0, i, 0)),
            pl.BlockSpec(w1n.shape, lambda i: (0, 0)),
            pl.BlockSpec(w1e.shape, lambda i: (0, 0, 0)),
            pl.BlockSpec(b1.shape, lambda i: (0,)),
        ],
        out_specs=[
            pl.BlockSpec((_BLK1, h), lambda i: (i, 0)),
            pl.BlockSpec((h,), lambda i: (0,)),
            pl.BlockSpec((h,), lambda i: (0,)),
        ],
        out_shape=[
            jax.ShapeDtypeStruct((b, h), jnp.float32),
            jax.ShapeDtypeStruct((h,), jnp.float32),
            jax.ShapeDtypeStruct((h,), jnp.float32),
        ],
        compiler_params=pltpu.CompilerParams(
            dimension_semantics=("arbitrary",)
        ),
    )(xn, emb_rows, w1n, w1e, b1)


def _bn_relu_matmul(z, s, q, g, beta, w, bias):
    b, h_out = z.shape[0], w.shape[1]

    def body(z_ref, s_ref, q_ref, g_ref, be_ref, w_ref, b_ref, z2_ref, s2_ref, q2_ref):
        mu = s_ref[...] * (1.0 / b)
        var = q_ref[...] * (1.0 / b) - mu * mu
        a = g_ref[...] * jax.lax.rsqrt(var + _EPS)
        c = be_ref[...] - a * mu
        h = jnp.maximum(z_ref[...] * a + c, 0.0)
        z2 = jnp.dot(h, w_ref[...], preferred_element_type=jnp.float32) + b_ref[...]
        z2_ref[...] = z2

        @pl.when(pl.program_id(0) == 0)
        def _():
            s2_ref[...] = jnp.zeros_like(s2_ref)
            q2_ref[...] = jnp.zeros_like(q2_ref)

        s2_ref[...] += jnp.sum(z2, axis=0)
        q2_ref[...] += jnp.sum(z2 * z2, axis=0)

    return pl.pallas_call(
        body,
        grid=(b // _BLK,),
        in_specs=[
            pl.BlockSpec((_BLK, z.shape[1]), lambda i: (i, 0)),
            pl.BlockSpec(s.shape, lambda i: (0,)),
            pl.BlockSpec(q.shape, lambda i: (0,)),
            pl.BlockSpec(g.shape, lambda i: (0,)),
            pl.BlockSpec(beta.shape, lambda i: (0,)),
            pl.BlockSpec(w.shape, lambda i: (0, 0)),
            pl.BlockSpec(bias.shape, lambda i: (0,)),
        ],
        out_specs=[
            pl.BlockSpec((_BLK, h_out), lambda i: (i, 0)),
            pl.BlockSpec((h_out,), lambda i: (0,)),
            pl.BlockSpec((h_out,), lambda i: (0,)),
        ],
        out_shape=[
            jax.ShapeDtypeStruct((b, h_out), jnp.float32),
            jax.ShapeDtypeStruct((h_out,), jnp.float32),
            jax.ShapeDtypeStruct((h_out,), jnp.float32),
        ],
        compiler_params=pltpu.CompilerParams(
            dimension_semantics=("arbitrary",)
        ),
    )(z, s, q, g, beta, w, bias)


def _final(z, s, q, g, beta, w4, b4):
    b = z.shape[0]

    def body(z_ref, s_ref, q_ref, g_ref, be_ref, w_ref, b_ref, o_ref):
        mu = s_ref[...] * (1.0 / b)
        var = q_ref[...] * (1.0 / b) - mu * mu
        a = g_ref[...] * jax.lax.rsqrt(var + _EPS)
        c = be_ref[...] - a * mu
        h = jnp.maximum(z_ref[...] * a + c, 0.0)
        logit = jnp.dot(h, w_ref[...], preferred_element_type=jnp.float32) + b_ref[...]
        o_ref[...] = jax.nn.sigmoid(logit)

    return pl.pallas_call(
        body,
        grid=(b // _BLK,),
        in_specs=[
            pl.BlockSpec((_BLK, z.shape[1]), lambda i: (i, 0)),
            pl.BlockSpec(s.shape, lambda i: (0,)),
            pl.BlockSpec(q.shape, lambda i: (0,)),
            pl.BlockSpec(g.shape, lambda i: (0,)),
            pl.BlockSpec(beta.shape, lambda i: (0,)),
            pl.BlockSpec(w4.shape, lambda i: (0, 0)),
            pl.BlockSpec(b4.shape, lambda i: (0,)),
        ],
        out_specs=pl.BlockSpec((_BLK, 1), lambda i: (i, 0)),
        out_shape=jax.ShapeDtypeStruct((b, 1), jnp.float32),
        compiler_params=pltpu.CompilerParams(
            dimension_semantics=("arbitrary",)
        ),
    )(z, s, q, g, beta, w4, b4)


def kernel(X_numerical, X_categorical, tables, W1, b1, g1, beta1, W2, b2, g2, beta2, W3, b3, g3, beta3, W4, b4):
    f, v, d = tables.shape
    b = X_numerical.shape[0]
    nnum = X_numerical.shape[1]

    t416 = tables.transpose(0, 2, 1).reshape(f * d, v)
    planes = _table_planes(t416, v)
    tables_flat = planes.reshape(_NPL * v * 8, d)

    xc = X_categorical.astype(jnp.int32)
    vcat_pm = jnp.stack(
        [xc[:, 0:8], xc[:, 8:16], xc[:, 16:24],
         jnp.concatenate([xc[:, 24:26], xc[:, 20:26]], axis=1)], axis=0)
    j_off = (jnp.arange(_NPL, dtype=jnp.int32) * (8 * v))[:, None, None]
    k_off = jnp.arange(8, dtype=jnp.int32)[None, None, :]
    flat_idx = (vcat_pm * 8 + j_off + k_off).reshape(1, b * _IPR)

    emb_rows = _sc_gather(tables_flat, flat_idx)
    emb4 = emb_rows.reshape(_NPL, b, 8 * d)

    w1n = W1[:nnum]
    w1e_pad = jnp.zeros((_IPR * d, W1.shape[1]), jnp.float32).at[: f * d].set(W1[nnum:])
    w1e4 = w1e_pad.reshape(_NPL, 8 * d, W1.shape[1])
    z1, s1, q1 = _layer1(X_numerical, emb4, w1n, w1e4, b1)
    z2, s2, q2 = _bn_relu_matmul(z1, s1, q1, g1, beta1, W2, b2)
    z3, s3, q3 = _bn_relu_matmul(z2, s2, q2, g2, beta2, W3, b3)
    out = _final(z3, s3, q3, g3, beta3, W4, b4)
    return out.reshape(b)


# speedup vs baseline: 5.0792x; 1.2446x over previous
import jax
import jax.numpy as jnp
from jax.experimental import pallas as pl
from jax.experimental.pallas import tpu as pltpu
from jax.experimental.pallas import tpu_sc as plsc

_EPS = 1e-5
_GW = 128
_IPR = 32
_NPL = 4


_VB = 4096


def _table_plane(t416, v, j):
    if j < 3:
        def body(a_ref, o_ref):
            o_ref[...] = a_ref[...].T

        in_spec = pl.BlockSpec((128, _VB), lambda i, j=j: (j, i))
    else:
        def body(a_ref, o_ref):
            t = a_ref[...].T
            o_ref[...] = jnp.concatenate(
                [t, jnp.zeros((_VB, 96), jnp.float32)], axis=1)

        in_spec = pl.BlockSpec((32, _VB), lambda i: (12, i))

    return pl.pallas_call(
        body,
        grid=((v + _VB - 1) // _VB,),
        in_specs=[in_spec],
        out_specs=pl.BlockSpec((_VB, 128), lambda i: (i, 0)),
        out_shape=jax.ShapeDtypeStruct((v, 128), jnp.float32),
        compiler_params=pltpu.CompilerParams(
            dimension_semantics=("arbitrary",)
        ),
    )(t416)


def _sc_gather(tables_flat, flat_idx):
    n = flat_idx.shape[1]
    d = tables_flat.shape[1]
    mesh = plsc.VectorSubcoreMesh(core_axis_name="core", subcore_axis_name="subcore")

    @pl.kernel(
        out_type=jax.ShapeDtypeStruct((n, d), tables_flat.dtype),
        mesh=mesh,
        compiler_params=pltpu.CompilerParams(use_tc_tiling_on_sc=False),
    )
    def gather_kernel(tab_hbm, idx_hbm, out_hbm):
        def body(i_vmem, o_vmem):
            pltpu.sync_copy(tab_hbm.at[i_vmem.at[0]], o_vmem)

        pltpu.emit_pipeline(
            body,
            grid=(n // _GW,),
            in_specs=[pl.BlockSpec((1, _GW), index_map=lambda i: (0, i))],
            out_specs=[pl.BlockSpec((_GW, d), index_map=lambda i: (i, 0))],
            core_axis_name=("core", "subcore"),
            dimension_semantics=(pltpu.PARALLEL,),
        )(idx_hbm, out_hbm)

    return gather_kernel(tables_flat, flat_idx)


_BLK = 2048


_BLK1 = 2048


def _layer1(xn, embs, w1n, w1e4, b1):
    b, h = xn.shape[0], w1n.shape[1]

    def body(xn_ref, e0, e1, e2, e3, wn_ref, we_ref, b_ref, z_ref, s_ref, q_ref):
        z = jnp.dot(xn_ref[...], wn_ref[...], preferred_element_type=jnp.float32)
        for j, e_ref in enumerate((e0, e1, e2, e3)):
            z = z + jnp.dot(e_ref[...], we_ref[j],
                            preferred_element_type=jnp.float32)
        z = z + b_ref[...]
        z_ref[...] = z

        @pl.when(pl.program_id(0) == 0)
        def _():
            s_ref[...] = jnp.zeros_like(s_ref)
            q_ref[...] = jnp.zeros_like(q_ref)

        s_ref[...] += jnp.sum(z, axis=0)
        q_ref[...] += jnp.sum(z * z, axis=0)

    return pl.pallas_call(
        body,
        grid=(b // _BLK1,),
        in_specs=[
            pl.BlockSpec((_BLK1, xn.shape[1]), lambda i: (i, 0)),
        ] + [
            pl.BlockSpec((_BLK1, 128), lambda i: (i, 0))
            for _ in range(_NPL)
        ] + [
            pl.BlockSpec(w1n.shape, lambda i: (0, 0)),
            pl.BlockSpec(w1e4.shape, lambda i: (0, 0, 0)),
            pl.BlockSpec(b1.shape, lambda i: (0,)),
        ],
        out_specs=[
            pl.BlockSpec((_BLK1, h), lambda i: (i, 0)),
            pl.BlockSpec((h,), lambda i: (0,)),
            pl.BlockSpec((h,), lambda i: (0,)),
        ],
        out_shape=[
            jax.ShapeDtypeStruct((b, h), jnp.float32),
            jax.ShapeDtypeStruct((h,), jnp.float32),
            jax.ShapeDtypeStruct((h,), jnp.float32),
        ],
        compiler_params=pltpu.CompilerParams(
            dimension_semantics=("arbitrary",)
        ),
    )(xn, *embs, w1n, w1e4, b1)


def _bn_relu_matmul(z, s, q, g, beta, w, bias):
    b, h_out = z.shape[0], w.shape[1]

    def body(z_ref, s_ref, q_ref, g_ref, be_ref, w_ref, b_ref, z2_ref, s2_ref, q2_ref):
        mu = s_ref[...] * (1.0 / b)
        var = q_ref[...] * (1.0 / b) - mu * mu
        a = g_ref[...] * jax.lax.rsqrt(var + _EPS)
        c = be_ref[...] - a * mu
        h = jnp.maximum(z_ref[...] * a + c, 0.0)
        z2 = jnp.dot(h, w_ref[...], preferred_element_type=jnp.float32) + b_ref[...]
        z2_ref[...] = z2

        @pl.when(pl.program_id(0) == 0)
        def _():
            s2_ref[...] = jnp.zeros_like(s2_ref)
            q2_ref[...] = jnp.zeros_like(q2_ref)

        s2_ref[...] += jnp.sum(z2, axis=0)
        q2_ref[...] += jnp.sum(z2 * z2, axis=0)

    return pl.pallas_call(
        body,
        grid=(b // _BLK,),
        in_specs=[
            pl.BlockSpec((_BLK, z.shape[1]), lambda i: (i, 0)),
            pl.BlockSpec(s.shape, lambda i: (0,)),
            pl.BlockSpec(q.shape, lambda i: (0,)),
            pl.BlockSpec(g.shape, lambda i: (0,)),
            pl.BlockSpec(beta.shape, lambda i: (0,)),
            pl.BlockSpec(w.shape, lambda i: (0, 0)),
            pl.BlockSpec(bias.shape, lambda i: (0,)),
        ],
        out_specs=[
            pl.BlockSpec((_BLK, h_out), lambda i: (i, 0)),
            pl.BlockSpec((h_out,), lambda i: (0,)),
            pl.BlockSpec((h_out,), lambda i: (0,)),
        ],
        out_shape=[
            jax.ShapeDtypeStruct((b, h_out), jnp.float32),
            jax.ShapeDtypeStruct((h_out,), jnp.float32),
            jax.ShapeDtypeStruct((h_out,), jnp.float32),
        ],
        compiler_params=pltpu.CompilerParams(
            dimension_semantics=("arbitrary",)
        ),
    )(z, s, q, g, beta, w, bias)


def _final(z, s, q, g, beta, w4, b4):
    b = z.shape[0]

    def body(z_ref, s_ref, q_ref, g_ref, be_ref, w_ref, b_ref, o_ref):
        mu = s_ref[...] * (1.0 / b)
        var = q_ref[...] * (1.0 / b) - mu * mu
        a = g_ref[...] * jax.lax.rsqrt(var + _EPS)
        c = be_ref[...] - a * mu
        h = jnp.maximum(z_ref[...] * a + c, 0.0)
        logit = jnp.dot(h, w_ref[...], preferred_element_type=jnp.float32) + b_ref[...]
        o_ref[...] = jax.nn.sigmoid(logit)

    return pl.pallas_call(
        body,
        grid=(b // _BLK,),
        in_specs=[
            pl.BlockSpec((_BLK, z.shape[1]), lambda i: (i, 0)),
            pl.BlockSpec(s.shape, lambda i: (0,)),
            pl.BlockSpec(q.shape, lambda i: (0,)),
            pl.BlockSpec(g.shape, lambda i: (0,)),
            pl.BlockSpec(beta.shape, lambda i: (0,)),
            pl.BlockSpec(w4.shape, lambda i: (0, 0)),
            pl.BlockSpec(b4.shape, lambda i: (0,)),
        ],
        out_specs=pl.BlockSpec((_BLK, 1), lambda i: (i, 0)),
        out_shape=jax.ShapeDtypeStruct((b, 1), jnp.float32),
        compiler_params=pltpu.CompilerParams(
            dimension_semantics=("arbitrary",)
        ),
    )(z, s, q, g, beta, w4, b4)


def kernel(X_numerical, X_categorical, tables, W1, b1, g1, beta1, W2, b2, g2, beta2, W3, b3, g3, beta3, W4, b4):
    f, v, d = tables.shape
    b = X_numerical.shape[0]
    nnum = X_numerical.shape[1]

    t416 = tables.transpose(0, 2, 1).reshape(f * d, v)

    xc = X_categorical.astype(jnp.int32)
    vcat_pm = [xc[:, 0:8], xc[:, 8:16], xc[:, 16:24],
               jnp.concatenate([xc[:, 24:26], xc[:, 20:26]], axis=1)]
    k_off = jnp.arange(8, dtype=jnp.int32)[None, :]

    embs = []
    for j in range(_NPL):
        plane = _table_plane(t416, v, j)
        idx_j = (vcat_pm[j] * 8 + k_off).reshape(1, b * 8)
        rows = _sc_gather(plane.reshape(v * 8, d), idx_j)
        embs.append(rows.reshape(b, 8 * d))

    w1n = W1[:nnum]
    w1e_pad = jnp.zeros((_IPR * d, W1.shape[1]), jnp.float32).at[: f * d].set(W1[nnum:])
    w1e4 = w1e_pad.reshape(_NPL, 8 * d, W1.shape[1])
    z1, s1, q1 = _layer1(X_numerical, embs, w1n, w1e4, b1)
    z2, s2, q2 = _bn_relu_matmul(z1, s1, q1, g1, beta1, W2, b2)
    z3, s3, q3 = _bn_relu_matmul(z2, s2, q2, g2, beta2, W3, b3)
    out = _final(z3, s3, q3, g3, beta3, W4, b4)
    return out.reshape(b)


# speedup vs baseline: 5.2951x; 1.0425x over previous
import jax
import jax.numpy as jnp
from jax.experimental import pallas as pl
from jax.experimental.pallas import tpu as pltpu
from jax.experimental.pallas import tpu_sc as plsc

_EPS = 1e-5
_GW = 128
_IPR = 32
_NPL = 4


_VB = 8192


def _table_plane(t416, v, j):
    if j < 3:
        def body(a_ref, o_ref):
            o_ref[...] = a_ref[...].T

        in_spec = pl.BlockSpec((128, _VB), lambda i, j=j: (j, i))
    else:
        def body(a_ref, o_ref):
            t = a_ref[...].T
            o_ref[...] = jnp.concatenate(
                [t, jnp.zeros((_VB, 96), jnp.float32)], axis=1)

        in_spec = pl.BlockSpec((32, _VB), lambda i: (12, i))

    return pl.pallas_call(
        body,
        grid=((v + _VB - 1) // _VB,),
        in_specs=[in_spec],
        out_specs=pl.BlockSpec((_VB, 128), lambda i: (i, 0)),
        out_shape=jax.ShapeDtypeStruct((v, 128), jnp.float32),
        compiler_params=pltpu.CompilerParams(
            dimension_semantics=("arbitrary",)
        ),
    )(t416)


def _sc_gather(tables_flat, flat_idx):
    n = flat_idx.shape[1]
    d = tables_flat.shape[1]
    mesh = plsc.VectorSubcoreMesh(core_axis_name="core", subcore_axis_name="subcore")

    @pl.kernel(
        out_type=jax.ShapeDtypeStruct((n, d), tables_flat.dtype),
        mesh=mesh,
        compiler_params=pltpu.CompilerParams(use_tc_tiling_on_sc=False),
    )
    def gather_kernel(tab_hbm, idx_hbm, out_hbm):
        def body(i_vmem, o_vmem):
            pltpu.sync_copy(tab_hbm.at[i_vmem.at[0]], o_vmem)

        pltpu.emit_pipeline(
            body,
            grid=(n // _GW,),
            in_specs=[pl.BlockSpec((1, _GW), index_map=lambda i: (0, i))],
            out_specs=[pl.BlockSpec((_GW, d), index_map=lambda i: (i, 0))],
            core_axis_name=("core", "subcore"),
            dimension_semantics=(pltpu.PARALLEL,),
        )(idx_hbm, out_hbm)

    return gather_kernel(tables_flat, flat_idx)


_BLK = 2048


_BLK1 = 2048


def _layer1(xn, embs, w1n, w1e4, b1):
    b, h = xn.shape[0], w1n.shape[1]

    def body(xn_ref, e0, e1, e2, e3, wn_ref, we_ref, b_ref, z_ref, s_ref, q_ref):
        z = jnp.dot(xn_ref[...], wn_ref[...], preferred_element_type=jnp.float32)
        for j, e_ref in enumerate((e0, e1, e2, e3)):
            z = z + jnp.dot(e_ref[...], we_ref[j],
                            preferred_element_type=jnp.float32)
        z = z + b_ref[...]
        z_ref[...] = z

        @pl.when(pl.program_id(0) == 0)
        def _():
            s_ref[...] = jnp.zeros_like(s_ref)
            q_ref[...] = jnp.zeros_like(q_ref)

        s_ref[...] += jnp.sum(z, axis=0)
        q_ref[...] += jnp.sum(z * z, axis=0)

    return pl.pallas_call(
        body,
        grid=(b // _BLK1,),
        in_specs=[
            pl.BlockSpec((_BLK1, xn.shape[1]), lambda i: (i, 0)),
        ] + [
            pl.BlockSpec((_BLK1, 128), lambda i: (i, 0))
            for _ in range(_NPL)
        ] + [
            pl.BlockSpec(w1n.shape, lambda i: (0, 0)),
            pl.BlockSpec(w1e4.shape, lambda i: (0, 0, 0)),
            pl.BlockSpec(b1.shape, lambda i: (0,)),
        ],
        out_specs=[
            pl.BlockSpec((_BLK1, h), lambda i: (i, 0)),
            pl.BlockSpec((h,), lambda i: (0,)),
            pl.BlockSpec((h,), lambda i: (0,)),
        ],
        out_shape=[
            jax.ShapeDtypeStruct((b, h), jnp.float32),
            jax.ShapeDtypeStruct((h,), jnp.float32),
            jax.ShapeDtypeStruct((h,), jnp.float32),
        ],
        compiler_params=pltpu.CompilerParams(
            dimension_semantics=("arbitrary",)
        ),
    )(xn, *embs, w1n, w1e4, b1)


def _bn_relu_matmul(z, s, q, g, beta, w, bias):
    b, h_out = z.shape[0], w.shape[1]

    def body(z_ref, s_ref, q_ref, g_ref, be_ref, w_ref, b_ref, z2_ref, s2_ref, q2_ref):
        mu = s_ref[...] * (1.0 / b)
        var = q_ref[...] * (1.0 / b) - mu * mu
        a = g_ref[...] * jax.lax.rsqrt(var + _EPS)
        c = be_ref[...] - a * mu
        h = jnp.maximum(z_ref[...] * a + c, 0.0)
        z2 = jnp.dot(h, w_ref[...], preferred_element_type=jnp.float32) + b_ref[...]
        z2_ref[...] = z2

        @pl.when(pl.program_id(0) == 0)
        def _():
            s2_ref[...] = jnp.zeros_like(s2_ref)
            q2_ref[...] = jnp.zeros_like(q2_ref)

        s2_ref[...] += jnp.sum(z2, axis=0)
        q2_ref[...] += jnp.sum(z2 * z2, axis=0)

    return pl.pallas_call(
        body,
        grid=(b // _BLK,),
        in_specs=[
            pl.BlockSpec((_BLK, z.shape[1]), lambda i: (i, 0)),
            pl.BlockSpec(s.shape, lambda i: (0,)),
            pl.BlockSpec(q.shape, lambda i: (0,)),
            pl.BlockSpec(g.shape, lambda i: (0,)),
            pl.BlockSpec(beta.shape, lambda i: (0,)),
            pl.BlockSpec(w.shape, lambda i: (0, 0)),
            pl.BlockSpec(bias.shape, lambda i: (0,)),
        ],
        out_specs=[
            pl.BlockSpec((_BLK, h_out), lambda i: (i, 0)),
            pl.BlockSpec((h_out,), lambda i: (0,)),
            pl.BlockSpec((h_out,), lambda i: (0,)),
        ],
        out_shape=[
            jax.ShapeDtypeStruct((b, h_out), jnp.float32),
            jax.ShapeDtypeStruct((h_out,), jnp.float32),
            jax.ShapeDtypeStruct((h_out,), jnp.float32),
        ],
        compiler_params=pltpu.CompilerParams(
            dimension_semantics=("arbitrary",)
        ),
    )(z, s, q, g, beta, w, bias)


def _final(z, s, q, g, beta, w4, b4):
    b = z.shape[0]

    def body(z_ref, s_ref, q_ref, g_ref, be_ref, w_ref, b_ref, o_ref):
        mu = s_ref[...] * (1.0 / b)
        var = q_ref[...] * (1.0 / b) - mu * mu
        a = g_ref[...] * jax.lax.rsqrt(var + _EPS)
        c = be_ref[...] - a * mu
        h = jnp.maximum(z_ref[...] * a + c, 0.0)
        logit = jnp.dot(h, w_ref[...], preferred_element_type=jnp.float32) + b_ref[...]
        o_ref[...] = jax.nn.sigmoid(logit)

    return pl.pallas_call(
        body,
        grid=(b // _BLK,),
        in_specs=[
            pl.BlockSpec((_BLK, z.shape[1]), lambda i: (i, 0)),
            pl.BlockSpec(s.shape, lambda i: (0,)),
            pl.BlockSpec(q.shape, lambda i: (0,)),
            pl.BlockSpec(g.shape, lambda i: (0,)),
            pl.BlockSpec(beta.shape, lambda i: (0,)),
            pl.BlockSpec(w4.shape, lambda i: (0, 0)),
            pl.BlockSpec(b4.shape, lambda i: (0,)),
        ],
        out_specs=pl.BlockSpec((_BLK, 1), lambda i: (i, 0)),
        out_shape=jax.ShapeDtypeStruct((b, 1), jnp.float32),
        compiler_params=pltpu.CompilerParams(
            dimension_semantics=("arbitrary",)
        ),
    )(z, s, q, g, beta, w4, b4)


def kernel(X_numerical, X_categorical, tables, W1, b1, g1, beta1, W2, b2, g2, beta2, W3, b3, g3, beta3, W4, b4):
    f, v, d = tables.shape
    b = X_numerical.shape[0]
    nnum = X_numerical.shape[1]

    t416 = tables.transpose(0, 2, 1).reshape(f * d, v)

    xc = X_categorical.astype(jnp.int32)
    vcat_pm = [xc[:, 0:8], xc[:, 8:16], xc[:, 16:24],
               jnp.concatenate([xc[:, 24:26], xc[:, 20:26]], axis=1)]
    k_off = jnp.arange(8, dtype=jnp.int32)[None, :]

    embs = []
    for j in range(_NPL):
        plane = _table_plane(t416, v, j)
        idx_j = (vcat_pm[j] * 8 + k_off).reshape(1, b * 8)
        rows = _sc_gather(plane.reshape(v * 8, d), idx_j)
        embs.append(rows.reshape(b, 8 * d))

    w1n = W1[:nnum]
    w1e_pad = jnp.zeros((_IPR * d, W1.shape[1]), jnp.float32).at[: f * d].set(W1[nnum:])
    w1e4 = w1e_pad.reshape(_NPL, 8 * d, W1.shape[1])
    z1, s1, q1 = _layer1(X_numerical, embs, w1n, w1e4, b1)
    z2, s2, q2 = _bn_relu_matmul(z1, s1, q1, g1, beta1, W2, b2)
    z3, s3, q3 = _bn_relu_matmul(z2, s2, q2, g2, beta2, W3, b3)
    out = _final(z3, s3, q3, g3, beta3, W4, b4)
    return out.reshape(b)


# speedup vs baseline: 5.2987x; 1.0007x over previous
import jax
import jax.numpy as jnp
from jax.experimental import pallas as pl
from jax.experimental.pallas import tpu as pltpu
from jax.experimental.pallas import tpu_sc as plsc

_EPS = 1e-5
_GW = 128
_IPR = 32
_NPL = 4


_VB = 16384


def _table_plane(t416, v, j):
    if j < 3:
        def body(a_ref, o_ref):
            o_ref[...] = a_ref[...].T

        in_spec = pl.BlockSpec((128, _VB), lambda i, j=j: (j, i))
    else:
        def body(a_ref, o_ref):
            t = a_ref[...].T
            o_ref[...] = jnp.concatenate(
                [t, jnp.zeros((_VB, 96), jnp.float32)], axis=1)

        in_spec = pl.BlockSpec((32, _VB), lambda i: (12, i))

    return pl.pallas_call(
        body,
        grid=((v + _VB - 1) // _VB,),
        in_specs=[in_spec],
        out_specs=pl.BlockSpec((_VB, 128), lambda i: (i, 0)),
        out_shape=jax.ShapeDtypeStruct((v, 128), jnp.float32),
        compiler_params=pltpu.CompilerParams(
            dimension_semantics=("arbitrary",)
        ),
    )(t416)


def _sc_gather(tables_flat, flat_idx):
    n = flat_idx.shape[1]
    d = tables_flat.shape[1]
    mesh = plsc.VectorSubcoreMesh(core_axis_name="core", subcore_axis_name="subcore")

    @pl.kernel(
        out_type=jax.ShapeDtypeStruct((n, d), tables_flat.dtype),
        mesh=mesh,
        compiler_params=pltpu.CompilerParams(use_tc_tiling_on_sc=False),
    )
    def gather_kernel(tab_hbm, idx_hbm, out_hbm):
        def body(i_vmem, o_vmem):
            pltpu.sync_copy(tab_hbm.at[i_vmem.at[0]], o_vmem)

        pltpu.emit_pipeline(
            body,
            grid=(n // _GW,),
            in_specs=[pl.BlockSpec((1, _GW), index_map=lambda i: (0, i))],
            out_specs=[pl.BlockSpec((_GW, d), index_map=lambda i: (i, 0))],
            core_axis_name=("core", "subcore"),
            dimension_semantics=(pltpu.PARALLEL,),
        )(idx_hbm, out_hbm)

    return gather_kernel(tables_flat, flat_idx)


_BLK = 2048


_BLK1 = 2048


def _layer1(xn, embs, w1n, w1e4, b1):
    b, h = xn.shape[0], w1n.shape[1]

    def body(xn_ref, e0, e1, e2, e3, wn_ref, we_ref, b_ref, z_ref, s_ref, q_ref):
        z = jnp.dot(xn_ref[...], wn_ref[...], preferred_element_type=jnp.float32)
        for j, e_ref in enumerate((e0, e1, e2, e3)):
            z = z + jnp.dot(e_ref[...], we_ref[j],
                            preferred_element_type=jnp.float32)
        z = z + b_ref[...]
        z_ref[...] = z

        @pl.when(pl.program_id(0) == 0)
        def _():
            s_ref[...] = jnp.zeros_like(s_ref)
            q_ref[...] = jnp.zeros_like(q_ref)

        s_ref[...] += jnp.sum(z, axis=0)
        q_ref[...] += jnp.sum(z * z, axis=0)

    return pl.pallas_call(
        body,
        grid=(b // _BLK1,),
        in_specs=[
            pl.BlockSpec((_BLK1, xn.shape[1]), lambda i: (i, 0)),
        ] + [
            pl.BlockSpec((_BLK1, 128), lambda i: (i, 0))
            for _ in range(_NPL)
        ] + [
            pl.BlockSpec(w1n.shape, lambda i: (0, 0)),
            pl.BlockSpec(w1e4.shape, lambda i: (0, 0, 0)),
            pl.BlockSpec(b1.shape, lambda i: (0,)),
        ],
        out_specs=[
            pl.BlockSpec((_BLK1, h), lambda i: (i, 0)),
            pl.BlockSpec((h,), lambda i: (0,)),
            pl.BlockSpec((h,), lambda i: (0,)),
        ],
        out_shape=[
            jax.ShapeDtypeStruct((b, h), jnp.float32),
            jax.ShapeDtypeStruct((h,), jnp.float32),
            jax.ShapeDtypeStruct((h,), jnp.float32),
        ],
        compiler_params=pltpu.CompilerParams(
            dimension_semantics=("arbitrary",)
        ),
    )(xn, *embs, w1n, w1e4, b1)


def _bn_relu_matmul(z, s, q, g, beta, w, bias):
    b, h_out = z.shape[0], w.shape[1]

    def body(z_ref, s_ref, q_ref, g_ref, be_ref, w_ref, b_ref, z2_ref, s2_ref, q2_ref):
        mu = s_ref[...] * (1.0 / b)
        var = q_ref[...] * (1.0 / b) - mu * mu
        a = g_ref[...] * jax.lax.rsqrt(var + _EPS)
        c = be_ref[...] - a * mu
        h = jnp.maximum(z_ref[...] * a + c, 0.0)
        z2 = jnp.dot(h, w_ref[...], preferred_element_type=jnp.float32) + b_ref[...]
        z2_ref[...] = z2

        @pl.when(pl.program_id(0) == 0)
        def _():
            s2_ref[...] = jnp.zeros_like(s2_ref)
            q2_ref[...] = jnp.zeros_like(q2_ref)

        s2_ref[...] += jnp.sum(z2, axis=0)
        q2_ref[...] += jnp.sum(z2 * z2, axis=0)

    return pl.pallas_call(
        body,
        grid=(b // _BLK,),
        in_specs=[
            pl.BlockSpec((_BLK, z.shape[1]), lambda i: (i, 0)),
            pl.BlockSpec(s.shape, lambda i: (0,)),
            pl.BlockSpec(q.shape, lambda i: (0,)),
            pl.BlockSpec(g.shape, lambda i: (0,)),
            pl.BlockSpec(beta.shape, lambda i: (0,)),
            pl.BlockSpec(w.shape, lambda i: (0, 0)),
            pl.BlockSpec(bias.shape, lambda i: (0,)),
        ],
        out_specs=[
            pl.BlockSpec((_BLK, h_out), lambda i: (i, 0)),
            pl.BlockSpec((h_out,), lambda i: (0,)),
            pl.BlockSpec((h_out,), lambda i: (0,)),
        ],
        out_shape=[
            jax.ShapeDtypeStruct((b, h_out), jnp.float32),
            jax.ShapeDtypeStruct((h_out,), jnp.float32),
            jax.ShapeDtypeStruct((h_out,), jnp.float32),
        ],
        compiler_params=pltpu.CompilerParams(
            dimension_semantics=("arbitrary",)
        ),
    )(z, s, q, g, beta, w, bias)


def _final(z, s, q, g, beta, w4, b4):
    b = z.shape[0]

    def body(z_ref, s_ref, q_ref, g_ref, be_ref, w_ref, b_ref, o_ref):
        mu = s_ref[...] * (1.0 / b)
        var = q_ref[...] * (1.0 / b) - mu * mu
        a = g_ref[...] * jax.lax.rsqrt(var + _EPS)
        c = be_ref[...] - a * mu
        h = jnp.maximum(z_ref[...] * a + c, 0.0)
        logit = jnp.dot(h, w_ref[...], preferred_element_type=jnp.float32) + b_ref[...]
        o_ref[...] = jax.nn.sigmoid(logit)

    return pl.pallas_call(
        body,
        grid=(b // _BLK,),
        in_specs=[
            pl.BlockSpec((_BLK, z.shape[1]), lambda i: (i, 0)),
            pl.BlockSpec(s.shape, lambda i: (0,)),
            pl.BlockSpec(q.shape, lambda i: (0,)),
            pl.BlockSpec(g.shape, lambda i: (0,)),
            pl.BlockSpec(beta.shape, lambda i: (0,)),
            pl.BlockSpec(w4.shape, lambda i: (0, 0)),
            pl.BlockSpec(b4.shape, lambda i: (0,)),
        ],
        out_specs=pl.BlockSpec((_BLK, 1), lambda i: (i, 0)),
        out_shape=jax.ShapeDtypeStruct((b, 1), jnp.float32),
        compiler_params=pltpu.CompilerParams(
            dimension_semantics=("arbitrary",)
        ),
    )(z, s, q, g, beta, w4, b4)


def kernel(X_numerical, X_categorical, tables, W1, b1, g1, beta1, W2, b2, g2, beta2, W3, b3, g3, beta3, W4, b4):
    f, v, d = tables.shape
    b = X_numerical.shape[0]
    nnum = X_numerical.shape[1]

    t416 = tables.transpose(0, 2, 1).reshape(f * d, v)

    xc = X_categorical.astype(jnp.int32)
    vcat_pm = [xc[:, 0:8], xc[:, 8:16], xc[:, 16:24],
               jnp.concatenate([xc[:, 24:26], xc[:, 20:26]], axis=1)]
    k_off = jnp.arange(8, dtype=jnp.int32)[None, :]

    embs = []
    for j in range(_NPL):
        plane = _table_plane(t416, v, j)
        idx_j = (vcat_pm[j] * 8 + k_off).reshape(1, b * 8)
        rows = _sc_gather(plane.reshape(v * 8, d), idx_j)
        embs.append(rows.reshape(b, 8 * d))

    w1n = W1[:nnum]
    w1e_pad = jnp.zeros((_IPR * d, W1.shape[1]), jnp.float32).at[: f * d].set(W1[nnum:])
    w1e4 = w1e_pad.reshape(_NPL, 8 * d, W1.shape[1])
    z1, s1, q1 = _layer1(X_numerical, embs, w1n, w1e4, b1)
    z2, s2, q2 = _bn_relu_matmul(z1, s1, q1, g1, beta1, W2, b2)
    z3, s3, q3 = _bn_relu_matmul(z2, s2, q2, g2, beta2, W3, b3)
    out = _final(z3, s3, q3, g3, beta3, W4, b4)
    return out.reshape(b)


# speedup vs baseline: 5.3040x; 1.0010x over previous
import jax
import jax.numpy as jnp
from jax.experimental import pallas as pl
from jax.experimental.pallas import tpu as pltpu
from jax.experimental.pallas import tpu_sc as plsc

_EPS = 1e-5
_GW = 128
_IPR = 32
_NPL = 4


_VB = 8192


def _table_plane(t416, v, j):
    if j < 3:
        def body(a_ref, o_ref):
            o_ref[...] = a_ref[...].T

        in_spec = pl.BlockSpec((128, _VB), lambda i, j=j: (j, i))
    else:
        def body(a_ref, o_ref):
            t = a_ref[...].T
            o_ref[...] = jnp.concatenate(
                [t, jnp.zeros((_VB, 96), jnp.float32)], axis=1)

        in_spec = pl.BlockSpec((32, _VB), lambda i: (12, i))

    return pl.pallas_call(
        body,
        grid=((v + _VB - 1) // _VB,),
        in_specs=[in_spec],
        out_specs=pl.BlockSpec((_VB, 128), lambda i: (i, 0)),
        out_shape=jax.ShapeDtypeStruct((v, 128), jnp.float32),
        compiler_params=pltpu.CompilerParams(
            dimension_semantics=("arbitrary",)
        ),
    )(t416)


def _sc_gather(tables_flat, flat_idx):
    n = flat_idx.shape[1]
    d = tables_flat.shape[1]
    mesh = plsc.VectorSubcoreMesh(core_axis_name="core", subcore_axis_name="subcore")

    @pl.kernel(
        out_type=jax.ShapeDtypeStruct((n, d), tables_flat.dtype),
        mesh=mesh,
        compiler_params=pltpu.CompilerParams(use_tc_tiling_on_sc=False),
    )
    def gather_kernel(tab_hbm, idx_hbm, out_hbm):
        def body(i_vmem, o_vmem):
            pltpu.sync_copy(tab_hbm.at[i_vmem.at[0]], o_vmem)

        pltpu.emit_pipeline(
            body,
            grid=(n // _GW,),
            in_specs=[pl.BlockSpec((1, _GW), index_map=lambda i: (0, i))],
            out_specs=[pl.BlockSpec((_GW, d), index_map=lambda i: (i, 0))],
            core_axis_name=("core", "subcore"),
            dimension_semantics=(pltpu.PARALLEL,),
        )(idx_hbm, out_hbm)

    return gather_kernel(tables_flat, flat_idx)


_BLK = 2048


_BLK1 = 2048


def _layer1(xn, embs, w1n, w1e4, b1):
    b, h = xn.shape[0], w1n.shape[1]

    def body(xn_ref, e0, e1, e2, e3, wn_ref, we_ref, b_ref, z_ref, s_ref, q_ref):
        z = jnp.dot(xn_ref[...], wn_ref[...], preferred_element_type=jnp.float32)
        for j, e_ref in enumerate((e0, e1, e2, e3)):
            z = z + jnp.dot(e_ref[...], we_ref[j],
                            preferred_element_type=jnp.float32)
        z = z + b_ref[...]
        z_ref[...] = z

        @pl.when(pl.program_id(0) == 0)
        def _():
            s_ref[...] = jnp.zeros_like(s_ref)
            q_ref[...] = jnp.zeros_like(q_ref)

        s_ref[...] += jnp.sum(z, axis=0)
        q_ref[...] += jnp.sum(z * z, axis=0)

    return pl.pallas_call(
        body,
        grid=(b // _BLK1,),
        in_specs=[
            pl.BlockSpec((_BLK1, xn.shape[1]), lambda i: (i, 0)),
        ] + [
            pl.BlockSpec((_BLK1, 128), lambda i: (i, 0))
            for _ in range(_NPL)
        ] + [
            pl.BlockSpec(w1n.shape, lambda i: (0, 0)),
            pl.BlockSpec(w1e4.shape, lambda i: (0, 0, 0)),
            pl.BlockSpec(b1.shape, lambda i: (0,)),
        ],
        out_specs=[
            pl.BlockSpec((_BLK1, h), lambda i: (i, 0)),
            pl.BlockSpec((h,), lambda i: (0,)),
            pl.BlockSpec((h,), lambda i: (0,)),
        ],
        out_shape=[
            jax.ShapeDtypeStruct((b, h), jnp.float32),
            jax.ShapeDtypeStruct((h,), jnp.float32),
            jax.ShapeDtypeStruct((h,), jnp.float32),
        ],
        compiler_params=pltpu.CompilerParams(
            dimension_semantics=("arbitrary",)
        ),
    )(xn, *embs, w1n, w1e4, b1)


def _bn_relu_matmul(z, s, q, g, beta, w, bias):
    b, h_out = z.shape[0], w.shape[1]

    def body(z_ref, s_ref, q_ref, g_ref, be_ref, w_ref, b_ref, z2_ref, s2_ref, q2_ref):
        mu = s_ref[...] * (1.0 / b)
        var = q_ref[...] * (1.0 / b) - mu * mu
        a = g_ref[...] * jax.lax.rsqrt(var + _EPS)
        c = be_ref[...] - a * mu
        h = jnp.maximum(z_ref[...] * a + c, 0.0)
        z2 = jnp.dot(h, w_ref[...], preferred_element_type=jnp.float32) + b_ref[...]
        z2_ref[...] = z2

        @pl.when(pl.program_id(0) == 0)
        def _():
            s2_ref[...] = jnp.zeros_like(s2_ref)
            q2_ref[...] = jnp.zeros_like(q2_ref)

        s2_ref[...] += jnp.sum(z2, axis=0)
        q2_ref[...] += jnp.sum(z2 * z2, axis=0)

    return pl.pallas_call(
        body,
        grid=(b // _BLK,),
        in_specs=[
            pl.BlockSpec((_BLK, z.shape[1]), lambda i: (i, 0)),
            pl.BlockSpec(s.shape, lambda i: (0,)),
            pl.BlockSpec(q.shape, lambda i: (0,)),
            pl.BlockSpec(g.shape, lambda i: (0,)),
            pl.BlockSpec(beta.shape, lambda i: (0,)),
            pl.BlockSpec(w.shape, lambda i: (0, 0)),
            pl.BlockSpec(bias.shape, lambda i: (0,)),
        ],
        out_specs=[
            pl.BlockSpec((_BLK, h_out), lambda i: (i, 0)),
            pl.BlockSpec((h_out,), lambda i: (0,)),
            pl.BlockSpec((h_out,), lambda i: (0,)),
        ],
        out_shape=[
            jax.ShapeDtypeStruct((b, h_out), jnp.float32),
            jax.ShapeDtypeStruct((h_out,), jnp.float32),
            jax.ShapeDtypeStruct((h_out,), jnp.float32),
        ],
        compiler_params=pltpu.CompilerParams(
            dimension_semantics=("arbitrary",)
        ),
    )(z, s, q, g, beta, w, bias)


def _final(z, s, q, g, beta, w4, b4):
    b = z.shape[0]

    def body(z_ref, s_ref, q_ref, g_ref, be_ref, w_ref, b_ref, o_ref):
        mu = s_ref[...] * (1.0 / b)
        var = q_ref[...] * (1.0 / b) - mu * mu
        a = g_ref[...] * jax.lax.rsqrt(var + _EPS)
        c = be_ref[...] - a * mu
        h = jnp.maximum(z_ref[...] * a + c, 0.0)
        logit = jnp.dot(h, w_ref[...], preferred_element_type=jnp.float32) + b_ref[...]
        o_ref[...] = jax.nn.sigmoid(logit)

    return pl.pallas_call(
        body,
        grid=(b // _BLK,),
        in_specs=[
            pl.BlockSpec((_BLK, z.shape[1]), lambda i: (i, 0)),
            pl.BlockSpec(s.shape, lambda i: (0,)),
            pl.BlockSpec(q.shape, lambda i: (0,)),
            pl.BlockSpec(g.shape, lambda i: (0,)),
            pl.BlockSpec(beta.shape, lambda i: (0,)),
            pl.BlockSpec(w4.shape, lambda i: (0, 0)),
            pl.BlockSpec(b4.shape, lambda i: (0,)),
        ],
        out_specs=pl.BlockSpec((_BLK, 1), lambda i: (i, 0)),
        out_shape=jax.ShapeDtypeStruct((b, 1), jnp.float32),
        compiler_params=pltpu.CompilerParams(
            dimension_semantics=("arbitrary",)
        ),
    )(z, s, q, g, beta, w4, b4)


def kernel(X_numerical, X_categorical, tables, W1, b1, g1, beta1, W2, b2, g2, beta2, W3, b3, g3, beta3, W4, b4):
    f, v, d = tables.shape
    b = X_numerical.shape[0]
    nnum = X_numerical.shape[1]

    t416 = tables.transpose(0, 2, 1).reshape(f * d, v)

    xc = X_categorical.astype(jnp.int32)
    vcat_pm = [xc[:, 0:8], xc[:, 8:16], xc[:, 16:24],
               jnp.concatenate([xc[:, 24:26], xc[:, 20:26]], axis=1)]
    k_off = jnp.arange(8, dtype=jnp.int32)[None, :]

    embs = []
    for j in range(_NPL):
        plane = _table_plane(t416, v, j)
        idx_j = (vcat_pm[j] * 8 + k_off).reshape(1, b * 8)
        rows = _sc_gather(plane.reshape(v * 8, d), idx_j)
        embs.append(rows.reshape(b, 8 * d))

    w1n = W1[:nnum]
    w1e_pad = jnp.zeros((_IPR * d, W1.shape[1]), jnp.float32).at[: f * d].set(W1[nnum:])
    w1e4 = w1e_pad.reshape(_NPL, 8 * d, W1.shape[1])
    z1, s1, q1 = _layer1(X_numerical, embs, w1n, w1e4, b1)
    z2, s2, q2 = _bn_relu_matmul(z1, s1, q1, g1, beta1, W2, b2)
    z3, s3, q3 = _bn_relu_matmul(z2, s2, q2, g2, beta2, W3, b3)
    out = _final(z3, s3, q3, g3, beta3, W4, b4)
    return out.reshape(b)


# speedup vs baseline: 5.5255x; 1.0418x over previous
import jax
import jax.numpy as jnp
from jax.experimental import pallas as pl
from jax.experimental.pallas import tpu as pltpu
from jax.experimental.pallas import tpu_sc as plsc

_EPS = 1e-5
_GW = 128
_IPR = 32
_NPL = 4


_VB = 8192


def _table_plane(t416, v, j):
    if j < 3:
        def body(a_ref, o_ref):
            o_ref[...] = a_ref[...].T

        in_spec = pl.BlockSpec((128, _VB), lambda i, j=j: (j, i))
    else:
        def body(a_ref, o_ref):
            t = a_ref[...].T
            o_ref[...] = jnp.concatenate(
                [t, jnp.zeros((_VB, 96), jnp.float32)], axis=1)

        in_spec = pl.BlockSpec((32, _VB), lambda i: (12, i))

    return pl.pallas_call(
        body,
        grid=((v + _VB - 1) // _VB,),
        in_specs=[in_spec],
        out_specs=pl.BlockSpec((_VB, 128), lambda i: (i, 0)),
        out_shape=jax.ShapeDtypeStruct((v, 128), jnp.float32),
        compiler_params=pltpu.CompilerParams(
            dimension_semantics=("arbitrary",)
        ),
    )(t416)


def _sc_gather(tables_flat, idx2d, row0, n):
    d = tables_flat.shape[1]
    mesh = plsc.VectorSubcoreMesh(core_axis_name="core", subcore_axis_name="subcore")

    @pl.kernel(
        out_type=jax.ShapeDtypeStruct((n, d), tables_flat.dtype),
        mesh=mesh,
        compiler_params=pltpu.CompilerParams(use_tc_tiling_on_sc=False),
    )
    def gather_kernel(tab_hbm, idx_hbm, out_hbm):
        def body(i_vmem, o_vmem):
            def inner(sem):
                h1 = pltpu.async_copy(
                    tab_hbm.at[i_vmem.at[0]], o_vmem.at[pl.ds(0, _GW)], sem)
                h2 = pltpu.async_copy(
                    tab_hbm.at[i_vmem.at[1]], o_vmem.at[pl.ds(_GW, _GW)], sem)
                h1.wait()
                h2.wait()

            pl.run_scoped(inner, pltpu.SemaphoreType.DMA)

        pltpu.emit_pipeline(
            body,
            grid=(n // (2 * _GW),),
            in_specs=[pl.BlockSpec((2, _GW), index_map=lambda i: (row0 // 2 + i, 0))],
            out_specs=[pl.BlockSpec((2 * _GW, d), index_map=lambda i: (i, 0))],
            core_axis_name=("core", "subcore"),
            dimension_semantics=(pltpu.PARALLEL,),
        )(idx_hbm, out_hbm)

    return gather_kernel(tables_flat, idx2d)


_BLK = 2048


_BLK1 = 2048


def _layer1(xn, embs, w1n, w1e4, b1):
    b, h = xn.shape[0], w1n.shape[1]

    def body(xn_ref, e0, e1, e2, e3, wn_ref, we_ref, b_ref, z_ref, s_ref, q_ref):
        z = jnp.dot(xn_ref[...], wn_ref[...], preferred_element_type=jnp.float32)
        for j, e_ref in enumerate((e0, e1, e2, e3)):
            z = z + jnp.dot(e_ref[...], we_ref[j],
                            preferred_element_type=jnp.float32)
        z = z + b_ref[...]
        z_ref[...] = z

        @pl.when(pl.program_id(0) == 0)
        def _():
            s_ref[...] = jnp.zeros_like(s_ref)
            q_ref[...] = jnp.zeros_like(q_ref)

        s_ref[...] += jnp.sum(z, axis=0)
        q_ref[...] += jnp.sum(z * z, axis=0)

    return pl.pallas_call(
        body,
        grid=(b // _BLK1,),
        in_specs=[
            pl.BlockSpec((_BLK1, xn.shape[1]), lambda i: (i, 0)),
        ] + [
            pl.BlockSpec((_BLK1, 128), lambda i: (i, 0))
            for _ in range(_NPL)
        ] + [
            pl.BlockSpec(w1n.shape, lambda i: (0, 0)),
            pl.BlockSpec(w1e4.shape, lambda i: (0, 0, 0)),
            pl.BlockSpec(b1.shape, lambda i: (0,)),
        ],
        out_specs=[
            pl.BlockSpec((_BLK1, h), lambda i: (i, 0)),
            pl.BlockSpec((h,), lambda i: (0,)),
            pl.BlockSpec((h,), lambda i: (0,)),
        ],
        out_shape=[
            jax.ShapeDtypeStruct((b, h), jnp.float32),
            jax.ShapeDtypeStruct((h,), jnp.float32),
            jax.ShapeDtypeStruct((h,), jnp.float32),
        ],
        compiler_params=pltpu.CompilerParams(
            dimension_semantics=("arbitrary",)
        ),
    )(xn, *embs, w1n, w1e4, b1)


def _bn_relu_matmul(z, s, q, g, beta, w, bias):
    b, h_out = z.shape[0], w.shape[1]

    def body(z_ref, s_ref, q_ref, g_ref, be_ref, w_ref, b_ref, z2_ref, s2_ref, q2_ref):
        mu = s_ref[...] * (1.0 / b)
        var = q_ref[...] * (1.0 / b) - mu * mu
        a = g_ref[...] * jax.lax.rsqrt(var + _EPS)
        c = be_ref[...] - a * mu
        h = jnp.maximum(z_ref[...] * a + c, 0.0)
        z2 = jnp.dot(h, w_ref[...], preferred_element_type=jnp.float32) + b_ref[...]
        z2_ref[...] = z2

        @pl.when(pl.program_id(0) == 0)
        def _():
            s2_ref[...] = jnp.zeros_like(s2_ref)
            q2_ref[...] = jnp.zeros_like(q2_ref)

        s2_ref[...] += jnp.sum(z2, axis=0)
        q2_ref[...] += jnp.sum(z2 * z2, axis=0)

    return pl.pallas_call(
        body,
        grid=(b // _BLK,),
        in_specs=[
            pl.BlockSpec((_BLK, z.shape[1]), lambda i: (i, 0)),
            pl.BlockSpec(s.shape, lambda i: (0,)),
            pl.BlockSpec(q.shape, lambda i: (0,)),
            pl.BlockSpec(g.shape, lambda i: (0,)),
            pl.BlockSpec(beta.shape, lambda i: (0,)),
            pl.BlockSpec(w.shape, lambda i: (0, 0)),
            pl.BlockSpec(bias.shape, lambda i: (0,)),
        ],
        out_specs=[
            pl.BlockSpec((_BLK, h_out), lambda i: (i, 0)),
            pl.BlockSpec((h_out,), lambda i: (0,)),
            pl.BlockSpec((h_out,), lambda i: (0,)),
        ],
        out_shape=[
            jax.ShapeDtypeStruct((b, h_out), jnp.float32),
            jax.ShapeDtypeStruct((h_out,), jnp.float32),
            jax.ShapeDtypeStruct((h_out,), jnp.float32),
        ],
        compiler_params=pltpu.CompilerParams(
            dimension_semantics=("arbitrary",)
        ),
    )(z, s, q, g, beta, w, bias)


def _final(z, s, q, g, beta, w4, b4):
    b = z.shape[0]

    def body(z_ref, s_ref, q_ref, g_ref, be_ref, w_ref, b_ref, o_ref):
        mu = s_ref[...] * (1.0 / b)
        var = q_ref[...] * (1.0 / b) - mu * mu
        a = g_ref[...] * jax.lax.rsqrt(var + _EPS)
        c = be_ref[...] - a * mu
        h = jnp.maximum(z_ref[...] * a + c, 0.0)
        logit = jnp.dot(h, w_ref[...], preferred_element_type=jnp.float32) + b_ref[...]
        o_ref[...] = jax.nn.sigmoid(logit)

    return pl.pallas_call(
        body,
        grid=(b // _BLK,),
        in_specs=[
            pl.BlockSpec((_BLK, z.shape[1]), lambda i: (i, 0)),
            pl.BlockSpec(s.shape, lambda i: (0,)),
            pl.BlockSpec(q.shape, lambda i: (0,)),
            pl.BlockSpec(g.shape, lambda i: (0,)),
            pl.BlockSpec(beta.shape, lambda i: (0,)),
            pl.BlockSpec(w4.shape, lambda i: (0, 0)),
            pl.BlockSpec(b4.shape, lambda i: (0,)),
        ],
        out_specs=pl.BlockSpec((_BLK, 1), lambda i: (i, 0)),
        out_shape=jax.ShapeDtypeStruct((b, 1), jnp.float32),
        compiler_params=pltpu.CompilerParams(
            dimension_semantics=("arbitrary",)
        ),
    )(z, s, q, g, beta, w4, b4)


def kernel(X_numerical, X_categorical, tables, W1, b1, g1, beta1, W2, b2, g2, beta2, W3, b3, g3, beta3, W4, b4):
    f, v, d = tables.shape
    b = X_numerical.shape[0]
    nnum = X_numerical.shape[1]

    t416 = tables.transpose(0, 2, 1).reshape(f * d, v)

    xc = X_categorical.astype(jnp.int32)
    vcat_pm = jnp.stack(
        [xc[:, 0:8], xc[:, 8:16], xc[:, 16:24],
         jnp.concatenate([xc[:, 24:26], xc[:, 20:26]], axis=1)], axis=0)
    k_off = jnp.arange(8, dtype=jnp.int32)[None, None, :]
    idx2d = (vcat_pm * 8 + k_off).reshape(_NPL * b * 8 // 128, 128)

    rows_per_plane = b * 8 // 128
    embs = [None] * _NPL
    for j in (3, 0, 1, 2):
        plane = _table_plane(t416, v, j)
        rows = _sc_gather(plane.reshape(v * 8, d), idx2d,
                          j * rows_per_plane, b * 8)
        embs[j] = rows.reshape(b, 8 * d)

    w1n = W1[:nnum]
    w1e_pad = jnp.zeros((_IPR * d, W1.shape[1]), jnp.float32).at[: f * d].set(W1[nnum:])
    w1e4 = w1e_pad.reshape(_NPL, 8 * d, W1.shape[1])
    z1, s1, q1 = _layer1(X_numerical, embs, w1n, w1e4, b1)
    z2, s2, q2 = _bn_relu_matmul(z1, s1, q1, g1, beta1, W2, b2)
    z3, s3, q3 = _bn_relu_matmul(z2, s2, q2, g2, beta2, W3, b3)
    out = _final(z3, s3, q3, g3, beta3, W4, b4)
    return out.reshape(b)


# speedup vs baseline: 5.7455x; 1.0398x over previous
import jax
import jax.numpy as jnp
from jax.experimental import pallas as pl
from jax.experimental.pallas import tpu as pltpu
from jax.experimental.pallas import tpu_sc as plsc

_EPS = 1e-5
_GW = 128
_IPR = 32
_NPL = 4


_VB = 8192


def _table_plane(t416, v, j):
    if j < 3:
        def body(a_ref, o_ref):
            o_ref[...] = a_ref[...].T

        in_spec = pl.BlockSpec((128, _VB), lambda i, j=j: (j, i))
    else:
        def body(a_ref, o_ref):
            t = a_ref[...].T
            o_ref[...] = jnp.concatenate(
                [t, jnp.zeros((_VB, 96), jnp.float32)], axis=1)

        in_spec = pl.BlockSpec((32, _VB), lambda i: (12, i))

    return pl.pallas_call(
        body,
        grid=((v + _VB - 1) // _VB,),
        in_specs=[in_spec],
        out_specs=pl.BlockSpec((_VB, 128), lambda i: (i, 0)),
        out_shape=jax.ShapeDtypeStruct((v, 128), jnp.float32),
        compiler_params=pltpu.CompilerParams(
            dimension_semantics=("arbitrary",)
        ),
    )(t416)


def _sc_gather(tables_flat, idx2d, row0, n):
    d = tables_flat.shape[1]
    mesh = plsc.VectorSubcoreMesh(core_axis_name="core", subcore_axis_name="subcore")

    @pl.kernel(
        out_type=jax.ShapeDtypeStruct((n, d), tables_flat.dtype),
        mesh=mesh,
        compiler_params=pltpu.CompilerParams(use_tc_tiling_on_sc=False),
    )
    def gather_kernel(tab_hbm, idx_hbm, out_hbm):
        def body(i_vmem, o_vmem):
            def inner(sem):
                hs = [pltpu.async_copy(
                    tab_hbm.at[i_vmem.at[w]], o_vmem.at[pl.ds(w * _GW, _GW)],
                    sem) for w in range(4)]
                for h in hs:
                    h.wait()

            pl.run_scoped(inner, pltpu.SemaphoreType.DMA)

        pltpu.emit_pipeline(
            body,
            grid=(n // (4 * _GW),),
            in_specs=[pl.BlockSpec((4, _GW), index_map=lambda i: (row0 // 4 + i, 0))],
            out_specs=[pl.BlockSpec((4 * _GW, d), index_map=lambda i: (i, 0))],
            core_axis_name=("core", "subcore"),
            dimension_semantics=(pltpu.PARALLEL,),
        )(idx_hbm, out_hbm)

    return gather_kernel(tables_flat, idx2d)


_BLK = 2048


_BLK1 = 2048


def _layer1(xn, embs, w1n, w1e4, b1):
    b, h = xn.shape[0], w1n.shape[1]

    def body(xn_ref, e0, e1, e2, e3, wn_ref, we_ref, b_ref, z_ref, s_ref, q_ref):
        z = jnp.dot(xn_ref[...], wn_ref[...], preferred_element_type=jnp.float32)
        for j, e_ref in enumerate((e0, e1, e2, e3)):
            z = z + jnp.dot(e_ref[...], we_ref[j],
                            preferred_element_type=jnp.float32)
        z = z + b_ref[...]
        z_ref[...] = z

        @pl.when(pl.program_id(0) == 0)
        def _():
            s_ref[...] = jnp.zeros_like(s_ref)
            q_ref[...] = jnp.zeros_like(q_ref)

        s_ref[...] += jnp.sum(z, axis=0)
        q_ref[...] += jnp.sum(z * z, axis=0)

    return pl.pallas_call(
        body,
        grid=(b // _BLK1,),
        in_specs=[
            pl.BlockSpec((_BLK1, xn.shape[1]), lambda i: (i, 0)),
        ] + [
            pl.BlockSpec((_BLK1, 128), lambda i: (i, 0))
            for _ in range(_NPL)
        ] + [
            pl.BlockSpec(w1n.shape, lambda i: (0, 0)),
            pl.BlockSpec(w1e4.shape, lambda i: (0, 0, 0)),
            pl.BlockSpec(b1.shape, lambda i: (0,)),
        ],
        out_specs=[
            pl.BlockSpec((_BLK1, h), lambda i: (i, 0)),
            pl.BlockSpec((h,), lambda i: (0,)),
            pl.BlockSpec((h,), lambda i: (0,)),
        ],
        out_shape=[
            jax.ShapeDtypeStruct((b, h), jnp.float32),
            jax.ShapeDtypeStruct((h,), jnp.float32),
            jax.ShapeDtypeStruct((h,), jnp.float32),
        ],
        compiler_params=pltpu.CompilerParams(
            dimension_semantics=("arbitrary",)
        ),
    )(xn, *embs, w1n, w1e4, b1)


def _bn_relu_matmul(z, s, q, g, beta, w, bias):
    b, h_out = z.shape[0], w.shape[1]

    def body(z_ref, s_ref, q_ref, g_ref, be_ref, w_ref, b_ref, z2_ref, s2_ref, q2_ref):
        mu = s_ref[...] * (1.0 / b)
        var = q_ref[...] * (1.0 / b) - mu * mu
        a = g_ref[...] * jax.lax.rsqrt(var + _EPS)
        c = be_ref[...] - a * mu
        h = jnp.maximum(z_ref[...] * a + c, 0.0)
        z2 = jnp.dot(h, w_ref[...], preferred_element_type=jnp.float32) + b_ref[...]
        z2_ref[...] = z2

        @pl.when(pl.program_id(0) == 0)
        def _():
            s2_ref[...] = jnp.zeros_like(s2_ref)
            q2_ref[...] = jnp.zeros_like(q2_ref)

        s2_ref[...] += jnp.sum(z2, axis=0)
        q2_ref[...] += jnp.sum(z2 * z2, axis=0)

    return pl.pallas_call(
        body,
        grid=(b // _BLK,),
        in_specs=[
            pl.BlockSpec((_BLK, z.shape[1]), lambda i: (i, 0)),
            pl.BlockSpec(s.shape, lambda i: (0,)),
            pl.BlockSpec(q.shape, lambda i: (0,)),
            pl.BlockSpec(g.shape, lambda i: (0,)),
            pl.BlockSpec(beta.shape, lambda i: (0,)),
            pl.BlockSpec(w.shape, lambda i: (0, 0)),
            pl.BlockSpec(bias.shape, lambda i: (0,)),
        ],
        out_specs=[
            pl.BlockSpec((_BLK, h_out), lambda i: (i, 0)),
            pl.BlockSpec((h_out,), lambda i: (0,)),
            pl.BlockSpec((h_out,), lambda i: (0,)),
        ],
        out_shape=[
            jax.ShapeDtypeStruct((b, h_out), jnp.float32),
            jax.ShapeDtypeStruct((h_out,), jnp.float32),
            jax.ShapeDtypeStruct((h_out,), jnp.float32),
        ],
        compiler_params=pltpu.CompilerParams(
            dimension_semantics=("arbitrary",)
        ),
    )(z, s, q, g, beta, w, bias)


def _final(z, s, q, g, beta, w4, b4):
    b = z.shape[0]

    def body(z_ref, s_ref, q_ref, g_ref, be_ref, w_ref, b_ref, o_ref):
        mu = s_ref[...] * (1.0 / b)
        var = q_ref[...] * (1.0 / b) - mu * mu
        a = g_ref[...] * jax.lax.rsqrt(var + _EPS)
        c = be_ref[...] - a * mu
        h = jnp.maximum(z_ref[...] * a + c, 0.0)
        logit = jnp.dot(h, w_ref[...], preferred_element_type=jnp.float32) + b_ref[...]
        o_ref[...] = jax.nn.sigmoid(logit)

    return pl.pallas_call(
        body,
        grid=(b // _BLK,),
        in_specs=[
            pl.BlockSpec((_BLK, z.shape[1]), lambda i: (i, 0)),
            pl.BlockSpec(s.shape, lambda i: (0,)),
            pl.BlockSpec(q.shape, lambda i: (0,)),
            pl.BlockSpec(g.shape, lambda i: (0,)),
            pl.BlockSpec(beta.shape, lambda i: (0,)),
            pl.BlockSpec(w4.shape, lambda i: (0, 0)),
            pl.BlockSpec(b4.shape, lambda i: (0,)),
        ],
        out_specs=pl.BlockSpec((_BLK, 1), lambda i: (i, 0)),
        out_shape=jax.ShapeDtypeStruct((b, 1), jnp.float32),
        compiler_params=pltpu.CompilerParams(
            dimension_semantics=("arbitrary",)
        ),
    )(z, s, q, g, beta, w4, b4)


def kernel(X_numerical, X_categorical, tables, W1, b1, g1, beta1, W2, b2, g2, beta2, W3, b3, g3, beta3, W4, b4):
    f, v, d = tables.shape
    b = X_numerical.shape[0]
    nnum = X_numerical.shape[1]

    t416 = tables.transpose(0, 2, 1).reshape(f * d, v)

    xc = X_categorical.astype(jnp.int32)
    vcat_pm = jnp.stack(
        [xc[:, 0:8], xc[:, 8:16], xc[:, 16:24],
         jnp.concatenate([xc[:, 24:26], xc[:, 20:26]], axis=1)], axis=0)
    k_off = jnp.arange(8, dtype=jnp.int32)[None, None, :]
    idx2d = (vcat_pm * 8 + k_off).reshape(_NPL * b * 8 // 128, 128)

    rows_per_plane = b * 8 // 128
    embs = [None] * _NPL
    for j in (3, 0, 1, 2):
        plane = _table_plane(t416, v, j)
        rows = _sc_gather(plane.reshape(v * 8, d), idx2d,
                          j * rows_per_plane, b * 8)
        embs[j] = rows.reshape(b, 8 * d)

    w1n = W1[:nnum]
    w1e_pad = jnp.zeros((_IPR * d, W1.shape[1]), jnp.float32).at[: f * d].set(W1[nnum:])
    w1e4 = w1e_pad.reshape(_NPL, 8 * d, W1.shape[1])
    z1, s1, q1 = _layer1(X_numerical, embs, w1n, w1e4, b1)
    z2, s2, q2 = _bn_relu_matmul(z1, s1, q1, g1, beta1, W2, b2)
    z3, s3, q3 = _bn_relu_matmul(z2, s2, q2, g2, beta2, W3, b3)
    out = _final(z3, s3, q3, g3, beta3, W4, b4)
    return out.reshape(b)
